# baseline (speedup 1.0000x reference)
import functools

import jax
import jax.numpy as jnp
from jax import lax
from jax.experimental import pallas as pl
from jax.experimental.pallas import tpu as pltpu
from jax.experimental.pallas import tpu_sc as plsc

_NUM_CORES = 2
_NUM_SUBCORES = 16
_NW = _NUM_CORES * _NUM_SUBCORES
_CHUNK = 64


@functools.partial(jax.jit, static_argnames=("n", "d"))
def _sc_embed(tok, table, pe2, *, n, d):
    per_w = n // _NW
    n_chunks = per_w // _CHUNK
    l_total = pe2.shape[0]

    mesh = plsc.VectorSubcoreMesh(
        core_axis_name="c", subcore_axis_name="s",
        num_cores=_NUM_CORES, num_subcores=_NUM_SUBCORES,
    )

    @functools.partial(
        pl.kernel,
        mesh=mesh,
        out_type=jax.ShapeDtypeStruct((n, d), jnp.float32),
        scratch_types=[
            pltpu.VMEM((_CHUNK,), jnp.int32),
            pltpu.VMEM((_CHUNK, d), jnp.float32),
            pltpu.VMEM((_CHUNK, d), jnp.float32),
            pltpu.SemaphoreType.DMA,
        ],
    )
    def k(tok_hbm, table_hbm, pe_hbm, out_hbm, idx_v, rows_v, pe_v, sem):
        wid = lax.axis_index("s") * _NUM_CORES + lax.axis_index("c")
        base = wid * per_w
        nvec = d // 16
        for c in range(n_chunks):
            row0 = base + c * _CHUNK
            l0 = lax.rem(row0, l_total)
            pltpu.sync_copy(tok_hbm.at[pl.ds(row0, _CHUNK)], idx_v)
            pltpu.sync_copy(pe_hbm.at[pl.ds(l0, _CHUNK), :], pe_v)
            pltpu.async_copy(table_hbm.at[idx_v], rows_v, sem).wait()

            def add_row(r, _):
                for j in range(nvec):
                    sl = pl.ds(j * 16, 16)
                    rows_v[r, sl] = rows_v[r, sl] + pe_v[r, sl]
                return _

            lax.fori_loop(0, _CHUNK, add_row, 0, unroll=False)
            pltpu.sync_copy(rows_v, out_hbm.at[pl.ds(row0, _CHUNK), :])

    return k(tok, table, pe2)


def kernel(tokens, table, pe):
    b, l = tokens.shape
    d = table.shape[1]
    tok = tokens.reshape(b * l)
    pe2 = pe.reshape(l, d)
    out = _sc_embed(tok, table, pe2, n=b * l, d=d)
    return out.reshape(b, l, d)


# speedup vs baseline: 1.2264x; 1.2264x over previous
import functools

import jax
import jax.numpy as jnp
from jax import lax
from jax.experimental import pallas as pl
from jax.experimental.pallas import tpu as pltpu
from jax.experimental.pallas import tpu_sc as plsc

_NUM_CORES = 2
_NUM_SUBCORES = 16
_NW = _NUM_CORES * _NUM_SUBCORES
_CHUNK = 32
_NROWBUF = 3
_NPEBUF = 2


@functools.partial(jax.jit, static_argnames=("n", "d"))
def _sc_embed(tok, table, pe2, *, n, d):
    per_w = n // _NW
    n_chunks = per_w // _CHUNK
    l_total = pe2.shape[0]
    nvec = d // 16

    mesh = plsc.VectorSubcoreMesh(
        core_axis_name="c", subcore_axis_name="s",
        num_cores=_NUM_CORES, num_subcores=_NUM_SUBCORES,
    )

    @functools.partial(
        pl.kernel,
        mesh=mesh,
        out_type=jax.ShapeDtypeStruct((n, d), jnp.float32),
        scratch_types=[
            pltpu.VMEM((per_w,), jnp.int32),
            [pltpu.VMEM((_CHUNK, d), jnp.float32) for _ in range(_NROWBUF)],
            [pltpu.VMEM((_CHUNK, d), jnp.float32) for _ in range(_NPEBUF)],
            [pltpu.SemaphoreType.DMA for _ in range(_NROWBUF)],
            [pltpu.SemaphoreType.DMA for _ in range(_NPEBUF)],
            [pltpu.SemaphoreType.DMA for _ in range(_NROWBUF)],
        ],
    )
    def k(tok_hbm, table_hbm, pe_hbm, out_hbm,
          idx_v, rows, peb, sem_g, sem_p, sem_s):
        wid = lax.axis_index("s") * _NUM_CORES + lax.axis_index("c")
        base = wid * per_w
        l0 = lax.rem(base, l_total)

        def gather(c):
            return pltpu.async_copy(
                table_hbm.at[idx_v.at[pl.ds(c * _CHUNK, _CHUNK)]],
                rows[c % _NROWBUF], sem_g[c % _NROWBUF])

        def pe_load(c):
            return pltpu.async_copy(
                pe_hbm.at[pl.ds(l0 + c * _CHUNK, _CHUNK), :],
                peb[c % _NPEBUF], sem_p[c % _NPEBUF])

        pltpu.sync_copy(tok_hbm.at[pl.ds(base, per_w)], idx_v)
        pend_g = {c: gather(c) for c in range(min(2, n_chunks))}
        pend_p = {c: pe_load(c) for c in range(min(2, n_chunks))}
        pend_s = {}

        for c in range(n_chunks):
            rb, pb = c % _NROWBUF, c % _NPEBUF
            if c + 2 < n_chunks:
                if c >= 1:
                    pend_s.pop(c - 1).wait()
                pend_g[c + 2] = gather(c + 2)
            pend_g.pop(c).wait()
            pend_p.pop(c).wait()

            def add_row(r, _):
                for j in range(nvec):
                    sl = pl.ds(j * 16, 16)
                    plsc.addupdate(rows[rb].at[r, sl], peb[pb][r, sl])
                return _

            lax.fori_loop(0, _CHUNK, add_row, 0, unroll=False)
            if c + 2 < n_chunks:
                pend_p[c + 2] = pe_load(c + 2)
            pend_s[c] = pltpu.async_copy(
                rows[rb], out_hbm.at[pl.ds(base + c * _CHUNK, _CHUNK), :],
                sem_s[rb])
        for c in sorted(pend_s):
            pend_s[c].wait()

    return k(tok, table, pe2)


def kernel(tokens, table, pe):
    b, l = tokens.shape
    d = table.shape[1]
    tok = tokens.reshape(b * l)
    pe2 = pe.reshape(l, d)
    out = _sc_embed(tok, table, pe2, n=b * l, d=d)
    return out.reshape(b, l, d)
